# baseline (speedup 1.0000x reference)
import jax
import jax.numpy as jnp
from jax.experimental import pallas as pl
from jax.experimental.pallas import tpu as pltpu

_BM = 400
_NBUF = 2


def _gconv_kernel(emb_ref, w_ref, bfc_ref, meta_ref, bias_ref, a_ref,
                  out_ref, feat_ref, buf_ref, copy_sem):
    i = pl.program_id(0)
    n_i = pl.num_programs(0)

    def copy_for(blk, slot):
        return pltpu.make_async_copy(
            meta_ref.at[pl.ds(blk * _BM, _BM), :],
            buf_ref.at[slot],
            copy_sem.at[slot],
        )

    @pl.when(i == 0)
    def _():
        copy_for(0, 0).start()
        copy_for(1, 1).start()
        feat_ref[...] = jax.lax.dot_general(
            emb_ref[...].astype(jnp.bfloat16),
            w_ref[...].astype(jnp.bfloat16),
            (((1,), (1,)), ((), ())),
            preferred_element_type=jnp.float32,
        ) + bfc_ref[...]

    slot = jax.lax.rem(i, _NBUF)
    copy_for(i, slot).wait()
    acc = jax.lax.dot_general(
        buf_ref[slot],
        feat_ref[...],
        (((1,), (0,)), ((), ())),
        precision=jax.lax.Precision.DEFAULT,
        preferred_element_type=jnp.float32,
    )
    r = acc + bias_ref[...]
    out_ref[...] = jnp.where(r >= 0, r, a_ref[0, 0] * r)

    @pl.when(i + _NBUF < n_i)
    def _():
        copy_for(i + _NBUF, slot).start()


def kernel(emb, meta, W, b_fc, bias, prelu_a):
    n, in_ch = emb.shape
    out_ch = W.shape[0]

    grid = (n // _BM,)
    out = pl.pallas_call(
        _gconv_kernel,
        grid=grid,
        in_specs=[
            pl.BlockSpec((n, in_ch), lambda i: (0, 0)),
            pl.BlockSpec((out_ch, in_ch), lambda i: (0, 0)),
            pl.BlockSpec((1, out_ch), lambda i: (0, 0)),
            pl.BlockSpec(memory_space=pltpu.MemorySpace.HBM),
            pl.BlockSpec((1, out_ch), lambda i: (0, 0)),
            pl.BlockSpec((1, 1), lambda i: (0, 0)),
        ],
        out_specs=pl.BlockSpec((_BM, out_ch), lambda i: (i, 0)),
        out_shape=jax.ShapeDtypeStruct((n, out_ch), jnp.float32),
        scratch_shapes=[
            pltpu.VMEM((n, out_ch), jnp.float32),
            pltpu.VMEM((_NBUF, _BM, n), jnp.float32),
            pltpu.SemaphoreType.DMA((_NBUF,)),
        ],
    )(emb, W, b_fc.reshape(1, out_ch), meta, bias.reshape(1, out_ch),
      prelu_a.reshape(1, 1))
    return out


# speedup vs baseline: 1.0068x; 1.0068x over previous
import jax
import jax.numpy as jnp
from jax.experimental import pallas as pl
from jax.experimental.pallas import tpu as pltpu


def _gconv_kernel(emb_ref, w_ref, bfc_ref, meta_ref, bias_ref, a_ref,
                  out_ref, feat_ref):
    @pl.when(pl.program_id(0) == 0)
    def _():
        acc = jax.lax.dot_general(
            emb_ref[...].astype(jnp.bfloat16),
            w_ref[...].astype(jnp.bfloat16),
            (((1,), (1,)), ((), ())),
            preferred_element_type=jnp.float32,
        )
        feat_ref[...] = acc + bfc_ref[...]

    acc = jax.lax.dot_general(
        meta_ref[...],
        feat_ref[...],
        (((1,), (0,)), ((), ())),
        precision=jax.lax.Precision.DEFAULT,
        preferred_element_type=jnp.float32,
    )
    r = acc + bias_ref[...]
    out_ref[...] = jnp.where(r >= 0, r, a_ref[0, 0] * r)


def kernel(emb, meta, W, b_fc, bias, prelu_a):
    n, in_ch = emb.shape
    out_ch = W.shape[0]

    bm = 400
    grid = (pl.cdiv(n, bm),)
    out = pl.pallas_call(
        _gconv_kernel,
        grid=grid,
        in_specs=[
            pl.BlockSpec((n, in_ch), lambda i: (0, 0)),
            pl.BlockSpec((out_ch, in_ch), lambda i: (0, 0)),
            pl.BlockSpec((1, out_ch), lambda i: (0, 0)),
            pl.BlockSpec((bm, n), lambda i: (i, 0)),
            pl.BlockSpec((1, out_ch), lambda i: (0, 0)),
            pl.BlockSpec((1, 1), lambda i: (0, 0)),
        ],
        out_specs=pl.BlockSpec((bm, out_ch), lambda i: (i, 0)),
        out_shape=jax.ShapeDtypeStruct((n, out_ch), jnp.float32),
        scratch_shapes=[pltpu.VMEM((n, out_ch), jnp.float32)],
    )(emb, W, b_fc.reshape(1, out_ch), meta, bias.reshape(1, out_ch),
      prelu_a.reshape(1, 1))
    return out
